# Initial kernel scaffold; baseline (speedup 1.0000x reference)
import jax
import jax.numpy as jnp
from jax.experimental import pallas as pl


def kernel(x, expert_ids, expert_weights):
    raise NotImplementedError("write your pallas kernel here")


# speedup vs baseline: 3.4531x; 3.4531x over previous
import functools

import jax
import jax.numpy as jnp
from jax import lax
from jax.experimental import pallas as pl
from jax.experimental.pallas import tpu as pltpu
from jax.experimental.pallas import tpu_sc as plsc

T = 32768
D = 1024
E = 64
CAP = 640
L = 16
NC, NS = 2, 16
NW = NC * NS
EPW = E // NW
CHT = 2048
NCH = T // CHT
RB = 32
PAD = 688


def _mesh():
    return plsc.VectorSubcoreMesh(core_axis_name="c", subcore_axis_name="s")


def _dispatch_body(x_hbm, eid_hbm, w_hbm, z_hbm,
                   dx_hbm, cw_hbm, ti_hbm, cnt_hbm,
                   eid_v, w_v, t0, c0, t1, c1, mg_i, mg_f, idx_v, rbuf, zb,
                   st_v):
    c = lax.axis_index("c")
    s = lax.axis_index("s")
    wid = s * NC + c
    e0 = wid * EPW
    iota = lax.iota(jnp.int32, L)
    e0v = jnp.broadcast_to(e0, (L,))
    e1v = e0v + 1

    pltpu.sync_copy(z_hbm, zb)

    def chunk_body(ch, carry):
        pltpu.sync_copy(eid_hbm.at[pl.ds(ch * CHT, CHT)], eid_v)
        pltpu.sync_copy(w_hbm.at[pl.ds(ch * CHT, CHT)], w_v)
        base_tok = ch * CHT

        def vreg_body(i, cc):
            cnt0, cnt1 = cc
            ev = eid_v[pl.ds(i * L, L)]
            wv = w_v[pl.ds(i * L, L)]
            tok = base_tok + i * L + iota

            def one(e_splat, cnt, t_ref, c_ref):
                m = ev == e_splat
                r = plsc.cumsum(jnp.where(m, 1, 0))
                pos = jnp.minimum(cnt + r - 1, PAD - 1)
                plsc.store_scatter(t_ref, [pos], tok, mask=m)
                plsc.store_scatter(c_ref, [pos], wv, mask=m)
                return cnt + jnp.max(r)

            cnt0 = one(e0v, cnt0, t0, c0)
            cnt1 = one(e1v, cnt1, t1, c1)
            return (cnt0, cnt1)

        return lax.fori_loop(0, CHT // L, vreg_body, carry)

    cnt0, cnt1 = lax.fori_loop(0, NCH, chunk_body,
                               (jnp.int32(0), jnp.int32(0)))

    st_v[...] = jnp.where(iota == 0, cnt0, jnp.where(iota == 1, cnt1, 0))
    pltpu.sync_copy(st_v, cnt_hbm.at[pl.ds(wid * L, L)])

    for j, (t_ref, c_ref, cnt) in enumerate(((t0, c0, cnt0), (t1, c1, cnt1))):
        e = e0 + j
        v = jnp.minimum(cnt, CAP)

        def mrow(k, _):
            sl = pl.ds(k * L, L)
            valid = (k * L + iota) < v
            mg_i[sl] = jnp.where(valid, t_ref[sl], -1)
            mg_f[sl] = jnp.where(valid, c_ref[sl], 0.0)
            return 0

        lax.fori_loop(0, CAP // L, mrow, 0)
        pltpu.sync_copy(mg_i, ti_hbm.at[pl.ds(pl.multiple_of(e * CAP, 8), CAP)])
        pltpu.sync_copy(mg_f, cw_hbm.at[pl.ds(pl.multiple_of(e * CAP, 8), CAP)])

        nfull = v // RB
        rem = v - nfull * RB

        def gchunk(i, _):
            w0 = pl.multiple_of(i * RB, 8)
            for q in range(RB // L):
                tv = t_ref[pl.ds(w0 + q * L, L)]
                idx_v[pl.ds(q * L, L)] = jnp.maximum(jnp.minimum(tv, T - 1), 0)
            pltpu.sync_copy(x_hbm.at[idx_v], rbuf)
            pltpu.sync_copy(rbuf, dx_hbm.at[e, pl.ds(w0, RB)])
            return 0

        lax.fori_loop(0, nfull, gchunk, 0)

        fl = pl.multiple_of(nfull * RB, 8)

        @pl.when(rem > 0)
        def _():
            for q in range(RB // L):
                pos = fl + q * L + iota
                tv = t_ref[pl.ds(fl + q * L, L)]
                tv = jnp.where(pos < v, tv, 0)
                idx_v[pl.ds(q * L, L)] = jnp.maximum(jnp.minimum(tv, T - 1), 0)
            pltpu.sync_copy(x_hbm.at[idx_v], rbuf)

            def zr(r, _):
                def zc(q2, _2):
                    rbuf[r, pl.ds(q2 * L, L)] = jnp.zeros((L,), jnp.float32)
                    return 0

                lax.fori_loop(0, D // L, zc, 0)
                return 0

            lax.fori_loop(rem, RB, zr, 0)
            pltpu.sync_copy(rbuf, dx_hbm.at[e, pl.ds(fl, RB)])

        fltot = fl + RB * jnp.minimum(rem, 1)
        nz = (CAP - fltot) // RB

        def zchunk(i, _):
            z = pl.multiple_of(fltot + i * RB, 8)
            pltpu.sync_copy(zb, dx_hbm.at[e, pl.ds(z, RB)])
            return 0

        lax.fori_loop(0, nz, zchunk, 0)


def _drops_body(cnt_hbm, out_hbm, cbuf, obuf):
    c = lax.axis_index("c")
    s = lax.axis_index("s")
    wid = s * NC + c

    @pl.when(wid == 0)
    def _():
        pltpu.sync_copy(cnt_hbm, cbuf)

        def body(i, acc):
            vv = cbuf[pl.ds(i * L, L)]
            return acc + jnp.maximum(vv - CAP, 0)

        acc = lax.fori_loop(0, NW, body, jnp.zeros((L,), jnp.int32))
        tot = jnp.sum(acc)
        obuf[...] = jnp.where(lax.iota(jnp.int32, L) == 0, tot, 0)
        pltpu.sync_copy(obuf, out_hbm)


def kernel(x, expert_ids, expert_weights):
    zeros = jnp.zeros((RB, D), jnp.float32)
    eid = expert_ids.astype(jnp.int32)

    k1 = pl.kernel(
        _dispatch_body,
        out_type=(
            jax.ShapeDtypeStruct((E, CAP, D), jnp.float32),
            jax.ShapeDtypeStruct((E * CAP,), jnp.float32),
            jax.ShapeDtypeStruct((E * CAP,), jnp.int32),
            jax.ShapeDtypeStruct((NW * L,), jnp.int32),
        ),
        mesh=_mesh(),
        compiler_params=pltpu.CompilerParams(needs_layout_passes=False),
        scratch_types=[
            pltpu.VMEM((CHT,), jnp.int32),
            pltpu.VMEM((CHT,), jnp.float32),
            pltpu.VMEM((PAD,), jnp.int32),
            pltpu.VMEM((PAD,), jnp.float32),
            pltpu.VMEM((PAD,), jnp.int32),
            pltpu.VMEM((PAD,), jnp.float32),
            pltpu.VMEM((CAP,), jnp.int32),
            pltpu.VMEM((CAP,), jnp.float32),
            pltpu.VMEM((RB,), jnp.int32),
            pltpu.VMEM((RB, D), jnp.float32),
            pltpu.VMEM((RB, D), jnp.float32),
            pltpu.VMEM((L,), jnp.int32),
        ],
    )
    dx, cw, ti, cnts = k1(x, eid, expert_weights, zeros)

    k2 = pl.kernel(
        _drops_body,
        out_type=jax.ShapeDtypeStruct((L,), jnp.int32),
        mesh=_mesh(),
        compiler_params=pltpu.CompilerParams(needs_layout_passes=False),
        scratch_types=[
            pltpu.VMEM((NW * L,), jnp.int32),
            pltpu.VMEM((L,), jnp.int32),
        ],
    )
    dropped = k2(cnts)[0]
    return dx, cw.reshape(E, CAP), ti.reshape(E, CAP), dropped


# speedup vs baseline: 4.1521x; 1.2024x over previous
import jax
import jax.numpy as jnp
from jax import lax
from jax.experimental import pallas as pl
from jax.experimental.pallas import tpu as pltpu
from jax.experimental.pallas import tpu_sc as plsc

T = 32768
D = 1024
E = 64
CAP = 640
L = 16
NC, NS = 2, 16
NW = NC * NS
EPW = E // NW
CHT = 2048
NCH = T // CHT
RB = 32
PAD = 688


def _mesh():
    return plsc.VectorSubcoreMesh(core_axis_name="c", subcore_axis_name="s")


def _dispatch_body(x_hbm, eid_hbm, w_hbm, z_hbm,
                   dx_hbm, cw_hbm, ti_hbm, cnt_hbm,
                   eid_a, eid_b, w_a, w_b, t0, c0, t1, c1, mg_i, mg_f,
                   idx_a, idx_b, rbuf_a, rbuf_b, zb, st_v,
                   lea, leb, lwa, lwb, ga, gb, zs):
    c = lax.axis_index("c")
    s = lax.axis_index("s")
    wid = s * NC + c
    e0 = wid * EPW
    iota = lax.iota(jnp.int32, L)
    e0v = jnp.broadcast_to(e0, (L,))
    e1v = e0v + 1

    pltpu.sync_copy(z_hbm, zb)

    def issue_load(ch, ebuf, wbuf, esem, wsem):
        pltpu.async_copy(eid_hbm.at[pl.ds(ch * CHT, CHT)], ebuf, esem)
        pltpu.async_copy(w_hbm.at[pl.ds(ch * CHT, CHT)], wbuf, wsem)

    def wait_load(ebuf, wbuf, esem, wsem):
        pltpu.make_async_copy(eid_hbm.at[pl.ds(0, CHT)], ebuf, esem).wait()
        pltpu.make_async_copy(w_hbm.at[pl.ds(0, CHT)], wbuf, wsem).wait()

    def scan_chunk(ch, ebuf, wbuf, carry):
        base_tok = ch * CHT

        def vreg_body(i, cc):
            cnt0, cnt1 = cc
            ev = ebuf[pl.ds(i * L, L)]
            wv = wbuf[pl.ds(i * L, L)]
            tok = base_tok + i * L + iota

            def one(e_splat, cnt, t_ref, c_ref):
                m = ev == e_splat
                r = plsc.cumsum(jnp.where(m, 1, 0))
                pos = jnp.minimum(cnt + r - 1, PAD - 1)
                plsc.store_scatter(t_ref, [pos], tok, mask=m)
                plsc.store_scatter(c_ref, [pos], wv, mask=m)
                return cnt + jnp.max(r)

            cnt0 = one(e0v, cnt0, t0, c0)
            cnt1 = one(e1v, cnt1, t1, c1)
            return (cnt0, cnt1)

        return lax.fori_loop(0, CHT // L, vreg_body, carry)

    issue_load(0, eid_a, w_a, lea, lwa)

    def pair_body(i2, carry):
        ch0 = 2 * i2
        issue_load(ch0 + 1, eid_b, w_b, leb, lwb)
        wait_load(eid_a, w_a, lea, lwa)
        carry = scan_chunk(ch0, eid_a, w_a, carry)

        @pl.when(ch0 + 2 < NCH)
        def _():
            issue_load(ch0 + 2, eid_a, w_a, lea, lwa)

        wait_load(eid_b, w_b, leb, lwb)
        return scan_chunk(ch0 + 1, eid_b, w_b, carry)

    cnt0, cnt1 = lax.fori_loop(0, NCH // 2, pair_body,
                               (jnp.int32(0), jnp.int32(0)))

    st_v[...] = jnp.where(iota == 0, cnt0, jnp.where(iota == 1, cnt1, 0))
    pltpu.sync_copy(st_v, cnt_hbm.at[pl.ds(wid * L, L)])

    for j, (t_ref, c_ref, cnt) in enumerate(((t0, c0, cnt0), (t1, c1, cnt1))):
        e = e0 + j
        v = jnp.minimum(cnt, CAP)

        def mrow(k, _):
            sl = pl.ds(k * L, L)
            valid = (k * L + iota) < v
            mg_i[sl] = jnp.where(valid, t_ref[sl], -1)
            mg_f[sl] = jnp.where(valid, c_ref[sl], 0.0)
            return 0

        lax.fori_loop(0, CAP // L, mrow, 0)
        pltpu.sync_copy(mg_i, ti_hbm.at[pl.ds(pl.multiple_of(e * CAP, 8), CAP)])
        pltpu.sync_copy(mg_f, cw_hbm.at[pl.ds(pl.multiple_of(e * CAP, 8), CAP)])

        nfull = v // RB
        rem = v - nfull * RB
        fl = pl.multiple_of(nfull * RB, 8)
        fltot = fl + RB * jnp.minimum(rem, 1)
        nz = (CAP - fltot) // RB

        def zchunk(i, _):
            z = pl.multiple_of(fltot + i * RB, 8)
            pltpu.async_copy(zb, dx_hbm.at[e, pl.ds(z, RB)], zs)
            return 0

        lax.fori_loop(0, nz, zchunk, 0)

        def build_idx(w0, ibuf):
            for q in range(RB // L):
                tv = t_ref[pl.ds(w0 + q * L, L)]
                ibuf[pl.ds(q * L, L)] = jnp.maximum(jnp.minimum(tv, T - 1), 0)

        def issue_gather(w0, ibuf, rb_, gsem):
            build_idx(w0, ibuf)
            pltpu.async_copy(x_hbm.at[ibuf], rb_, gsem)

        def wait_gather(ibuf, rb_, gsem):
            pltpu.make_async_copy(x_hbm.at[ibuf], rb_, gsem).wait()

        @pl.when(nfull > 0)
        def _():
            issue_gather(0, idx_a, rbuf_a, ga)

        def gpair(i2, _):
            c0_ = pl.multiple_of(2 * i2 * RB, 8)
            c1_ = c0_ + RB

            @pl.when(c1_ < fl)
            def _():
                issue_gather(c1_, idx_b, rbuf_b, gb)

            wait_gather(idx_a, rbuf_a, ga)
            pltpu.sync_copy(rbuf_a, dx_hbm.at[e, pl.ds(c0_, RB)])

            @pl.when(c0_ + 2 * RB < fl)
            def _():
                issue_gather(c0_ + 2 * RB, idx_a, rbuf_a, ga)

            @pl.when(c1_ < fl)
            def _():
                wait_gather(idx_b, rbuf_b, gb)
                pltpu.sync_copy(rbuf_b, dx_hbm.at[e, pl.ds(c1_, RB)])

            return 0

        lax.fori_loop(0, (nfull + 1) // 2, gpair, 0)

        @pl.when(rem > 0)
        def _():
            for q in range(RB // L):
                pos = fl + q * L + iota
                tv = t_ref[pl.ds(fl + q * L, L)]
                tv = jnp.where(pos < v, tv, 0)
                idx_a[pl.ds(q * L, L)] = jnp.maximum(jnp.minimum(tv, T - 1), 0)
            pltpu.sync_copy(x_hbm.at[idx_a], rbuf_a)

            def zr(r, _):
                def zc(q2, _2):
                    rbuf_a[r, pl.ds(q2 * L, L)] = jnp.zeros((L,), jnp.float32)
                    return 0

                lax.fori_loop(0, D // L, zc, 0)
                return 0

            lax.fori_loop(rem, RB, zr, 0)
            pltpu.sync_copy(rbuf_a, dx_hbm.at[e, pl.ds(fl, RB)])

        def zdrain(i, _):
            pltpu.make_async_copy(zb, dx_hbm.at[e, pl.ds(0, RB)], zs).wait()
            return 0

        lax.fori_loop(0, nz, zdrain, 0)


def _drops_body(cnt_hbm, out_hbm, cbuf, obuf):
    c = lax.axis_index("c")
    s = lax.axis_index("s")
    wid = s * NC + c

    @pl.when(wid == 0)
    def _():
        pltpu.sync_copy(cnt_hbm, cbuf)

        def body(i, acc):
            vv = cbuf[pl.ds(i * L, L)]
            return acc + jnp.maximum(vv - CAP, 0)

        acc = lax.fori_loop(0, NW, body, jnp.zeros((L,), jnp.int32))
        tot = jnp.sum(acc)
        obuf[...] = jnp.where(lax.iota(jnp.int32, L) == 0, tot, 0)
        pltpu.sync_copy(obuf, out_hbm)


def kernel(x, expert_ids, expert_weights):
    zeros = jnp.zeros((RB, D), jnp.float32)
    eid = expert_ids.astype(jnp.int32)

    k1 = pl.kernel(
        _dispatch_body,
        out_type=(
            jax.ShapeDtypeStruct((E, CAP, D), jnp.float32),
            jax.ShapeDtypeStruct((E * CAP,), jnp.float32),
            jax.ShapeDtypeStruct((E * CAP,), jnp.int32),
            jax.ShapeDtypeStruct((NW * L,), jnp.int32),
        ),
        mesh=_mesh(),
        compiler_params=pltpu.CompilerParams(needs_layout_passes=False),
        scratch_types=[
            pltpu.VMEM((CHT,), jnp.int32),
            pltpu.VMEM((CHT,), jnp.int32),
            pltpu.VMEM((CHT,), jnp.float32),
            pltpu.VMEM((CHT,), jnp.float32),
            pltpu.VMEM((PAD,), jnp.int32),
            pltpu.VMEM((PAD,), jnp.float32),
            pltpu.VMEM((PAD,), jnp.int32),
            pltpu.VMEM((PAD,), jnp.float32),
            pltpu.VMEM((CAP,), jnp.int32),
            pltpu.VMEM((CAP,), jnp.float32),
            pltpu.VMEM((RB,), jnp.int32),
            pltpu.VMEM((RB,), jnp.int32),
            pltpu.VMEM((RB, D), jnp.float32),
            pltpu.VMEM((RB, D), jnp.float32),
            pltpu.VMEM((RB, D), jnp.float32),
            pltpu.VMEM((L,), jnp.int32),
            pltpu.SemaphoreType.DMA,
            pltpu.SemaphoreType.DMA,
            pltpu.SemaphoreType.DMA,
            pltpu.SemaphoreType.DMA,
            pltpu.SemaphoreType.DMA,
            pltpu.SemaphoreType.DMA,
            pltpu.SemaphoreType.DMA,
        ],
    )
    dx, cw, ti, cnts = k1(x, eid, expert_weights, zeros)

    k2 = pl.kernel(
        _drops_body,
        out_type=jax.ShapeDtypeStruct((L,), jnp.int32),
        mesh=_mesh(),
        compiler_params=pltpu.CompilerParams(needs_layout_passes=False),
        scratch_types=[
            pltpu.VMEM((NW * L,), jnp.int32),
            pltpu.VMEM((L,), jnp.int32),
        ],
    )
    dropped = k2(cnts)[0]
    return dx, cw.reshape(E, CAP), ti.reshape(E, CAP), dropped
